# baseline (speedup 1.0000x reference)
import jax
import jax.numpy as jnp
from jax import lax
from jax.experimental import pallas as pl
from jax.experimental.pallas import tpu as pltpu

_TOP_K = 2
_NEG_INF = float("-inf")

def _router_body(xc_ref, wr_ref, br_ref, wn_ref, bn_ref, noise_ref,
                 idx_ref, gate_ref, pooled_ref):
    b = pl.program_id(0)
    nsteps = pl.num_programs(0)
    pooled_ref[pl.ds(b, 1), :] = jnp.mean(xc_ref[0], axis=1)[None, :]

    @pl.when(b == nsteps - 1)
    def _():
        pooled = pooled_ref[...]
        logits = jnp.dot(pooled, wr_ref[...],
                         preferred_element_type=jnp.float32) + br_ref[0]
        nlog = jnp.dot(pooled, wn_ref[...],
                       preferred_element_type=jnp.float32) + bn_ref[0]
        noisy = logits + noise_ref[...] * jax.nn.softplus(nlog)

        bs, ne = noisy.shape
        eids = lax.broadcasted_iota(jnp.int32, (bs, ne), 1)
        v0 = jnp.max(noisy, axis=1)
        i0 = jnp.min(jnp.where(noisy == v0[:, None], eids, ne), axis=1)
        masked = jnp.where(eids == i0[:, None], _NEG_INF, noisy)
        v1 = jnp.max(masked, axis=1)
        i1 = jnp.min(jnp.where(masked == v1[:, None], eids, ne), axis=1)
        t = jnp.exp(v1 - v0)
        g0 = 1.0 / (1.0 + t)
        g1 = t / (1.0 + t)
        idx_ref[...] = jnp.concatenate([i0[None, :], i1[None, :]], axis=0)
        gate_ref[...] = jnp.concatenate([g0[None, :], g1[None, :]], axis=0)


def _expert_body(idx_ref, gate_ref, xc_ref, w1_ref, b1_ref, w2_ref, b2_ref,
                 out_ref):
    b = pl.program_id(0)
    e0 = idx_ref[0, b]
    e1 = idx_ref[1, b]
    g0 = gate_ref[0, b]
    g1 = gate_ref[1, b]
    hid = b1_ref.shape[1]
    dim = b2_ref.shape[1]
    b1a = b1_ref[pl.ds(e0, 1), :].reshape(hid, 1)
    b1b = b1_ref[pl.ds(e1, 1), :].reshape(hid, 1)
    b2a = b2_ref[pl.ds(e0, 1), :].reshape(dim, 1)
    b2b = b2_ref[pl.ds(e1, 1), :].reshape(dim, 1)
    xb = xc_ref[0]
    cdim = (((0,), (0,)), ((), ()))
    h1a = jnp.maximum(
        lax.dot_general(w1_ref[e0], xb, cdim,
                        preferred_element_type=jnp.float32)
        + b1a, 0.0)
    h1b = jnp.maximum(
        lax.dot_general(w1_ref[e1], xb, cdim,
                        preferred_element_type=jnp.float32)
        + b1b, 0.0)
    h2a = lax.dot_general(w2_ref[e0], h1a, cdim,
                          preferred_element_type=jnp.float32)
    h2b = lax.dot_general(w2_ref[e1], h1b, cdim,
                          preferred_element_type=jnp.float32)
    out_ref[0] = g0 * (h2a + b2a) + g1 * (h2b + b2b)


def kernel(x, Wr, br, Wn, bn, W1, b1, W2, b2):
    bs, dim, h, w = x.shape
    hw = h * w
    ne = Wr.shape[1]
    hid = W1.shape[2]

    xc = x.reshape(bs, dim, hw)
    noise = jax.random.normal(jax.random.key(42), (bs, ne), dtype=jnp.float32)

    idx, gates = pl.pallas_call(
        _router_body,
        grid=(bs,),
        in_specs=[
            pl.BlockSpec((1, dim, hw), lambda b: (b, 0, 0)),
            pl.BlockSpec((dim, ne), lambda b: (0, 0)),
            pl.BlockSpec((1, ne), lambda b: (0, 0)),
            pl.BlockSpec((dim, ne), lambda b: (0, 0)),
            pl.BlockSpec((1, ne), lambda b: (0, 0)),
            pl.BlockSpec((bs, ne), lambda b: (0, 0)),
        ],
        out_specs=(
            pl.BlockSpec((_TOP_K, bs), lambda b: (0, 0)),
            pl.BlockSpec((_TOP_K, bs), lambda b: (0, 0)),
        ),
        out_shape=(
            jax.ShapeDtypeStruct((_TOP_K, bs), jnp.int32),
            jax.ShapeDtypeStruct((_TOP_K, bs), jnp.float32),
        ),
        scratch_shapes=[pltpu.VMEM((bs, dim), jnp.float32)],
    )(xc, Wr, br.reshape(1, ne), Wn, bn.reshape(1, ne), noise)

    def _e0(b, i_ref, g_ref):
        return (i_ref[0, b], 0, 0)

    def _e1(b, i_ref, g_ref):
        return (i_ref[1, b], 0, 0)

    grid_spec = pltpu.PrefetchScalarGridSpec(
        num_scalar_prefetch=2,
        grid=(bs,),
        in_specs=[
            pl.BlockSpec((1, dim, hw), lambda b, i_ref, g_ref: (b, 0, 0)),
            pl.BlockSpec((ne, dim, hid), lambda b, i_ref, g_ref: (0, 0, 0)),
            pl.BlockSpec((ne, hid), lambda b, i_ref, g_ref: (0, 0)),
            pl.BlockSpec((ne, hid, dim), lambda b, i_ref, g_ref: (0, 0, 0)),
            pl.BlockSpec((ne, dim), lambda b, i_ref, g_ref: (0, 0)),
        ],
        out_specs=pl.BlockSpec((1, dim, hw), lambda b, i_ref, g_ref: (b, 0, 0)),
    )
    outp = pl.pallas_call(
        _expert_body,
        grid_spec=grid_spec,
        out_shape=jax.ShapeDtypeStruct((bs, dim, hw), jnp.float32),
    )(idx, gates, xc, W1, b1, W2, b2)

    return outp.reshape(bs, dim, h, w)


# speedup vs baseline: 1.0996x; 1.0996x over previous
import jax
import jax.numpy as jnp
from jax import lax
from jax.experimental import pallas as pl
from jax.experimental.pallas import tpu as pltpu

_TOP_K = 2
_NEG_INF = float("-inf")
_IMGS_PER_STEP = 2


def _router_body(xc_ref, wr_ref, br_ref, wn_ref, bn_ref, noise_ref,
                 idx_ref, gate_ref):
    pooled = jnp.mean(xc_ref[...], axis=2)
    logits = jnp.dot(pooled, wr_ref[...],
                     preferred_element_type=jnp.float32) + br_ref[0]
    nlog = jnp.dot(pooled, wn_ref[...],
                   preferred_element_type=jnp.float32) + bn_ref[0]
    noisy = logits + noise_ref[...] * jax.nn.softplus(nlog)

    bs, ne = noisy.shape
    eids = lax.broadcasted_iota(jnp.int32, (bs, ne), 1)
    v0 = jnp.max(noisy, axis=1)
    i0 = jnp.min(jnp.where(noisy == v0[:, None], eids, ne), axis=1)
    masked = jnp.where(eids == i0[:, None], _NEG_INF, noisy)
    v1 = jnp.max(masked, axis=1)
    i1 = jnp.min(jnp.where(masked == v1[:, None], eids, ne), axis=1)
    t = jnp.exp(v1 - v0)
    g0 = 1.0 / (1.0 + t)
    g1 = t / (1.0 + t)
    idx_ref[...] = jnp.concatenate([i0[None, :], i1[None, :]], axis=0)
    gate_ref[...] = jnp.concatenate([g0[None, :], g1[None, :]], axis=0)


def _expert_body(idx_ref, gate_ref, xc_ref, w1_ref, b1_ref, w2_ref, b2_ref,
                 out_ref):
    s = pl.program_id(0)
    hid = b1_ref.shape[1]
    dim = b2_ref.shape[1]
    cdim = (((0,), (0,)), ((), ()))
    for j in range(_IMGS_PER_STEP):
        col = _IMGS_PER_STEP * s + j
        e0 = idx_ref[0, col]
        e1 = idx_ref[1, col]
        g0 = gate_ref[0, col]
        g1 = gate_ref[1, col]
        b1a = b1_ref[pl.ds(e0, 1), :].reshape(hid, 1)
        b1b = b1_ref[pl.ds(e1, 1), :].reshape(hid, 1)
        b2a = b2_ref[pl.ds(e0, 1), :].reshape(dim, 1)
        b2b = b2_ref[pl.ds(e1, 1), :].reshape(dim, 1)
        xb = xc_ref[j]
        h1a = jnp.maximum(
            lax.dot_general(w1_ref[e0], xb, cdim,
                            preferred_element_type=jnp.float32)
            + b1a, 0.0)
        h1b = jnp.maximum(
            lax.dot_general(w1_ref[e1], xb, cdim,
                            preferred_element_type=jnp.float32)
            + b1b, 0.0)
        h2a = lax.dot_general(w2_ref[e0], h1a, cdim,
                              preferred_element_type=jnp.float32)
        h2b = lax.dot_general(w2_ref[e1], h1b, cdim,
                              preferred_element_type=jnp.float32)
        out_ref[j] = g0 * (h2a + b2a) + g1 * (h2b + b2b)


def kernel(x, Wr, br, Wn, bn, W1, b1, W2, b2):
    bs, dim, h, w = x.shape
    hw = h * w
    ne = Wr.shape[1]
    hid = W1.shape[2]

    xc = x.reshape(bs, dim, hw)
    noise = jax.random.normal(jax.random.key(42), (bs, ne), dtype=jnp.float32)

    idx, gates = pl.pallas_call(
        _router_body,
        out_shape=(
            jax.ShapeDtypeStruct((_TOP_K, bs), jnp.int32),
            jax.ShapeDtypeStruct((_TOP_K, bs), jnp.float32),
        ),
    )(xc, Wr, br.reshape(1, ne), Wn, bn.reshape(1, ne), noise)

    nsteps = bs // _IMGS_PER_STEP
    grid_spec = pltpu.PrefetchScalarGridSpec(
        num_scalar_prefetch=2,
        grid=(nsteps,),
        in_specs=[
            pl.BlockSpec((_IMGS_PER_STEP, dim, hw),
                         lambda s, i_ref, g_ref: (s, 0, 0)),
            pl.BlockSpec((ne, dim, hid), lambda s, i_ref, g_ref: (0, 0, 0)),
            pl.BlockSpec((ne, hid), lambda s, i_ref, g_ref: (0, 0)),
            pl.BlockSpec((ne, hid, dim), lambda s, i_ref, g_ref: (0, 0, 0)),
            pl.BlockSpec((ne, dim), lambda s, i_ref, g_ref: (0, 0)),
        ],
        out_specs=pl.BlockSpec((_IMGS_PER_STEP, dim, hw),
                               lambda s, i_ref, g_ref: (s, 0, 0)),
    )
    outp = pl.pallas_call(
        _expert_body,
        grid_spec=grid_spec,
        out_shape=jax.ShapeDtypeStruct((bs, dim, hw), jnp.float32),
    )(idx, gates, xc, W1, b1, W2, b2)

    return outp.reshape(bs, dim, h, w)


# speedup vs baseline: 1.2285x; 1.1172x over previous
import jax
import jax.numpy as jnp
from jax import lax
from jax.experimental import pallas as pl
from jax.experimental.pallas import tpu as pltpu

_N_EMBED = 192
_NUM_EXPERTS = 8
_TOP_K = 2
_NEG_INF = float("-inf")


def _router_body(xt_ref, wr_ref, br_ref, wn_ref, bn_ref, noise_ref,
                 idx_ref, gate_ref):
    xs = xt_ref[...]
    pooled = jnp.mean(xs, axis=1)
    logits = jnp.dot(pooled, wr_ref[...],
                     preferred_element_type=jnp.float32) + br_ref[0]
    nlog = jnp.dot(pooled, wn_ref[...],
                   preferred_element_type=jnp.float32) + bn_ref[0]
    noisy = logits + noise_ref[...] * jax.nn.softplus(nlog)

    bs, ne = noisy.shape
    eids = lax.broadcasted_iota(jnp.int32, (bs, ne), 1)
    v0 = jnp.max(noisy, axis=1)
    i0 = jnp.min(jnp.where(noisy == v0[:, None], eids, ne), axis=1)
    masked = jnp.where(eids == i0[:, None], _NEG_INF, noisy)
    v1 = jnp.max(masked, axis=1)
    i1 = jnp.min(jnp.where(masked == v1[:, None], eids, ne), axis=1)
    t = jnp.exp(v1 - v0)
    g0 = 1.0 / (1.0 + t)
    g1 = t / (1.0 + t)
    idx_ref[...] = jnp.concatenate([i0[:, None], i1[:, None]], axis=1)
    gate_ref[...] = jnp.concatenate([g0[:, None], g1[:, None]], axis=1)


def _expert_body(idx_ref, gate_ref, xt_ref, w1_ref, b1_ref, w2_ref, b2_ref,
                 out_ref):
    b = pl.program_id(0)
    k = pl.program_id(1)
    e = idx_ref[2 * b + k]
    g = gate_ref[2 * b + k]
    xb = xt_ref[0]
    h1 = jnp.dot(xb, w1_ref[0], preferred_element_type=jnp.float32)
    h1 = jnp.maximum(h1 + b1_ref[pl.ds(e, 1), :], 0.0)
    h2 = jnp.dot(h1, w2_ref[0], preferred_element_type=jnp.float32)
    val = g * (h2 + b2_ref[pl.ds(e, 1), :])

    @pl.when(k == 0)
    def _():
        out_ref[0] = val

    @pl.when(k != 0)
    def _():
        out_ref[0] += val


def kernel(x, Wr, br, Wn, bn, W1, b1, W2, b2):
    bs, dim, h, w = x.shape
    hw = h * w
    ne = Wr.shape[1]
    hid = W1.shape[2]

    xt = jnp.transpose(x, (0, 2, 3, 1)).reshape(bs, hw, dim)
    noise = jax.random.normal(jax.random.key(42), (bs, ne), dtype=jnp.float32)

    idx, gates = pl.pallas_call(
        _router_body,
        out_shape=(
            jax.ShapeDtypeStruct((bs, _TOP_K), jnp.int32),
            jax.ShapeDtypeStruct((bs, _TOP_K), jnp.float32),
        ),
    )(xt, Wr, br.reshape(1, ne), Wn, bn.reshape(1, ne), noise)

    idx_flat = idx.reshape(bs * _TOP_K)
    gates_flat = gates.reshape(bs * _TOP_K)

    grid_spec = pltpu.PrefetchScalarGridSpec(
        num_scalar_prefetch=2,
        grid=(bs, _TOP_K),
        in_specs=[
            pl.BlockSpec((1, hw, dim), lambda b, k, i_ref, g_ref: (b, 0, 0)),
            pl.BlockSpec((1, dim, hid),
                         lambda b, k, i_ref, g_ref: (i_ref[2 * b + k], 0, 0)),
            pl.BlockSpec((ne, hid), lambda b, k, i_ref, g_ref: (0, 0)),
            pl.BlockSpec((1, hid, dim),
                         lambda b, k, i_ref, g_ref: (i_ref[2 * b + k], 0, 0)),
            pl.BlockSpec((ne, dim), lambda b, k, i_ref, g_ref: (0, 0)),
        ],
        out_specs=pl.BlockSpec((1, hw, dim), lambda b, k, i_ref, g_ref: (b, 0, 0)),
    )
    outp = pl.pallas_call(
        _expert_body,
        grid_spec=grid_spec,
        out_shape=jax.ShapeDtypeStruct((bs, hw, dim), jnp.float32),
    )(idx_flat, gates_flat, xt, W1, b1, W2, b2)

    return jnp.transpose(outp, (0, 2, 1)).reshape(bs, dim, h, w)


# speedup vs baseline: 1.3244x; 1.0781x over previous
import jax
import jax.numpy as jnp
from jax import lax
from jax.experimental import pallas as pl
from jax.experimental.pallas import tpu as pltpu

_N_EMBED = 192
_NUM_EXPERTS = 8
_TOP_K = 2
_NEG_INF = float("-inf")


def _router_body(xt_ref, wr_ref, br_ref, wn_ref, bn_ref, noise_ref,
                 idx_ref, gate_ref):
    xs = xt_ref[...]
    pooled = jnp.mean(xs, axis=1)
    logits = jnp.dot(pooled, wr_ref[...],
                     preferred_element_type=jnp.float32) + br_ref[0]
    nlog = jnp.dot(pooled, wn_ref[...],
                   preferred_element_type=jnp.float32) + bn_ref[0]
    noisy = logits + noise_ref[...] * jax.nn.softplus(nlog)

    bs, ne = noisy.shape
    eids = lax.broadcasted_iota(jnp.int32, (bs, ne), 1)
    v0 = jnp.max(noisy, axis=1)
    i0 = jnp.min(jnp.where(noisy == v0[:, None], eids, ne), axis=1)
    masked = jnp.where(eids == i0[:, None], _NEG_INF, noisy)
    v1 = jnp.max(masked, axis=1)
    i1 = jnp.min(jnp.where(masked == v1[:, None], eids, ne), axis=1)
    t = jnp.exp(v1 - v0)
    g0 = 1.0 / (1.0 + t)
    g1 = t / (1.0 + t)
    idx_ref[...] = jnp.concatenate([i0[None, :], i1[None, :]], axis=0)
    gate_ref[...] = jnp.concatenate([g0[None, :], g1[None, :]], axis=0)


def _expert_body(idx_ref, gate_ref, xt_ref, w1_ref, b1_ref, w2_ref, b2_ref,
                 out_ref):
    b = pl.program_id(0)
    k = pl.program_id(1)
    e = idx_ref[k, b]
    g = gate_ref[k, b]
    xb = xt_ref[0]
    h1 = jnp.dot(xb, w1_ref[0], preferred_element_type=jnp.float32)
    h1 = jnp.maximum(h1 + b1_ref[pl.ds(e, 1), :], 0.0)
    h2 = jnp.dot(h1, w2_ref[0], preferred_element_type=jnp.float32)
    val = g * (h2 + b2_ref[pl.ds(e, 1), :])

    @pl.when(k == 0)
    def _():
        out_ref[0] = val

    @pl.when(k != 0)
    def _():
        out_ref[0] += val


def kernel(x, Wr, br, Wn, bn, W1, b1, W2, b2):
    bs, dim, h, w = x.shape
    hw = h * w
    ne = Wr.shape[1]
    hid = W1.shape[2]

    xt = jnp.transpose(x, (0, 2, 3, 1)).reshape(bs, hw, dim)
    noise = jax.random.normal(jax.random.key(42), (bs, ne), dtype=jnp.float32)

    idx, gates = pl.pallas_call(
        _router_body,
        out_shape=(
            jax.ShapeDtypeStruct((_TOP_K, bs), jnp.int32),
            jax.ShapeDtypeStruct((_TOP_K, bs), jnp.float32),
        ),
    )(xt, Wr, br.reshape(1, ne), Wn, bn.reshape(1, ne), noise)

    grid_spec = pltpu.PrefetchScalarGridSpec(
        num_scalar_prefetch=2,
        grid=(bs, _TOP_K),
        in_specs=[
            pl.BlockSpec((1, hw, dim), lambda b, k, i_ref, g_ref: (b, 0, 0)),
            pl.BlockSpec((1, dim, hid),
                         lambda b, k, i_ref, g_ref: (i_ref[k, b], 0, 0)),
            pl.BlockSpec((ne, hid), lambda b, k, i_ref, g_ref: (0, 0)),
            pl.BlockSpec((1, hid, dim),
                         lambda b, k, i_ref, g_ref: (i_ref[k, b], 0, 0)),
            pl.BlockSpec((ne, dim), lambda b, k, i_ref, g_ref: (0, 0)),
        ],
        out_specs=pl.BlockSpec((1, hw, dim), lambda b, k, i_ref, g_ref: (b, 0, 0)),
    )
    outp = pl.pallas_call(
        _expert_body,
        grid_spec=grid_spec,
        out_shape=jax.ShapeDtypeStruct((bs, hw, dim), jnp.float32),
    )(idx, gates, xt, W1, b1, W2, b2)

    return jnp.transpose(outp, (0, 2, 1)).reshape(bs, dim, h, w)
